# Initial kernel scaffold; baseline (speedup 1.0000x reference)
import jax
import jax.numpy as jnp
from jax.experimental import pallas as pl


def kernel(all_ids, mem_input, ts, mem_ts, mem, h, num_dst_nodes, time_w, time_b, w_ih, w_hh, b_ih, b_hh, proj_w, proj_b):
    raise NotImplementedError("write your pallas kernel here")


# speedup vs baseline: 1.0666x; 1.0666x over previous
import functools

import jax
import jax.numpy as jnp
from jax import lax
from jax.experimental import pallas as pl
from jax.experimental.pallas import tpu as pltpu
from jax.experimental.pallas import tpu_sc as plsc

N = 100000
XW = 512
CNT_COL = 383
NPASS = 16
CH = 125
NSUB = 16
NCORE = 2
ROWS_PER_TILE = N // NSUB
NCHUNK_S = ROWS_PER_TILE // CH
ROWS_PER_W = N // (NSUB * NCORE)
NCHUNK_G = ROWS_PER_W // CH
ZROWS = 625
BR = 1000


def _build_body(mi_ref, ts_ref, mts_ref, tw_ref, tb_ref, mem_ref, x_ref):
    dt = ts_ref[...] - mts_ref[...]
    tf = jnp.cos(dt * tw_ref[...] + tb_ref[...])
    br = mi_ref.shape[0]
    x_ref[...] = jnp.concatenate([
        mi_ref[...], tf,
        jnp.zeros((br, 11), jnp.float32),
        jnp.ones((br, 1), jnp.float32),
        mem_ref[...],
    ], axis=1)


def _gru_body(t_ref, wih_ref, whh_ref, bih_ref, bhh_ref, upd_ref):
    tb = t_ref[...]
    xa = tb[:, :384]
    ma = tb[:, 384:]
    cnt = tb[:, CNT_COL:CNT_COL + 1]
    hprev = ma / jnp.maximum(cnt, 1.0)
    gi = jnp.dot(xa, wih_ref[...], preferred_element_type=jnp.float32) + bih_ref[...]
    gh = jnp.dot(hprev, whh_ref[...], preferred_element_type=jnp.float32) + bhh_ref[...]
    r = jax.nn.sigmoid(gi[:, :128] + gh[:, :128])
    z = jax.nn.sigmoid(gi[:, 128:256] + gh[:, 128:256])
    n = jnp.tanh(gi[:, 256:] + r * gh[:, 256:])
    upd_ref[...] = (1.0 - z) * n + z * hprev


def _final_body(r_ref, h_ref, pw_ref, pb_ref, o_ref):
    o_ref[...] = (r_ref[...]
                  + jnp.dot(h_ref[...], pw_ref[...], preferred_element_type=jnp.float32)
                  + pb_ref[...])


_sc_mesh = plsc.VectorSubcoreMesh(core_axis_name="c", subcore_axis_name="s")
_sc_params = pltpu.CompilerParams(use_tc_tiling_on_sc=False)


@functools.partial(
    pl.kernel,
    out_type=jax.ShapeDtypeStruct((N, XW), jnp.float32),
    mesh=_sc_mesh,
    compiler_params=_sc_params,
    scratch_types=[
        pltpu.VMEM_SHARED((N, 16), jnp.float32),
        pltpu.VMEM((NCHUNK_S, CH), jnp.int32),
        pltpu.VMEM((CH, 16), jnp.float32),
        pltpu.VMEM((ZROWS, 16), jnp.float32),
    ],
)
def _scatter_kernel(x_hbm, ids_hbm, table_hbm, acc, ids_v, buf, zbuf):
    c = lax.axis_index("c")
    s = lax.axis_index("s")
    row0 = s * ROWS_PER_TILE
    pltpu.sync_copy(ids_hbm.at[s], ids_v)

    def zrow(i, carry):
        zbuf[i, :] = jnp.zeros((16,), jnp.float32)
        return carry
    lax.fori_loop(0, ZROWS, zrow, 0)

    for p in range(NPASS):
        col0 = (p * NCORE + c) * 16

        def zcopy(i, carry):
            pltpu.sync_copy(zbuf, acc.at[pl.ds(row0 + i * ZROWS, ZROWS), :])
            return carry
        lax.fori_loop(0, ROWS_PER_TILE // ZROWS, zcopy, 0)
        plsc.subcore_barrier()

        def chunk(j, carry):
            pltpu.sync_copy(
                x_hbm.at[pl.ds(row0 + j * CH, CH), pl.ds(col0, 16)], buf)
            pltpu.sync_copy(buf, acc.at[ids_v.at[j]], add=True)
            return carry
        lax.fori_loop(0, NCHUNK_S, chunk, 0)
        plsc.subcore_barrier()

        pltpu.sync_copy(
            acc.at[pl.ds(row0, ROWS_PER_TILE), :],
            table_hbm.at[pl.ds(row0, ROWS_PER_TILE), pl.ds(col0, 16)])
        plsc.subcore_barrier()


@functools.partial(
    pl.kernel,
    out_type=jax.ShapeDtypeStruct((N, 128), jnp.float32),
    mesh=_sc_mesh,
    compiler_params=_sc_params,
    scratch_types=[
        pltpu.VMEM((NCHUNK_G, CH), jnp.int32),
        pltpu.VMEM((CH, 128), jnp.float32),
        pltpu.SemaphoreType.DMA,
    ],
)
def _gather_kernel(upd_hbm, ids_hbm, out_hbm, ids_v, rows_v, sem):
    c = lax.axis_index("c")
    s = lax.axis_index("s")
    w = s * NCORE + c
    chunk0 = w * NCHUNK_G
    pltpu.sync_copy(ids_hbm.at[w], ids_v)

    def chunk(j, carry):
        pltpu.async_copy(upd_hbm.at[ids_v.at[j]], rows_v, sem).wait()
        pltpu.sync_copy(rows_v, out_hbm.at[pl.ds((chunk0 + j) * CH, CH), :])
        return carry
    lax.fori_loop(0, NCHUNK_G, chunk, 0)


def kernel(all_ids, mem_input, ts, mem_ts, mem, h, num_dst_nodes,
           time_w, time_b, w_ih, w_hh, b_ih, b_hh, proj_w, proj_b):
    ids_i32 = all_ids.astype(jnp.int32)
    ids_s = ids_i32.reshape(NSUB, NCHUNK_S, CH)
    ids_g = ids_i32.reshape(NSUB * NCORE, NCHUNK_G, CH)

    x = pl.pallas_call(
        _build_body,
        grid=(N // BR,),
        in_specs=[
            pl.BlockSpec((BR, 272), lambda i: (i, 0)),
            pl.BlockSpec((BR, 1), lambda i: (i, 0)),
            pl.BlockSpec((BR, 1), lambda i: (i, 0)),
            pl.BlockSpec((1, 100), lambda i: (0, 0)),
            pl.BlockSpec((1, 100), lambda i: (0, 0)),
            pl.BlockSpec((BR, 128), lambda i: (i, 0)),
        ],
        out_specs=pl.BlockSpec((BR, XW), lambda i: (i, 0)),
        out_shape=jax.ShapeDtypeStruct((N, XW), jnp.float32),
    )(mem_input, ts.reshape(N, 1), mem_ts.reshape(N, 1),
      time_w.reshape(1, 100), time_b.reshape(1, 100), mem)

    table = _scatter_kernel(x, ids_s)

    wih_pad = jnp.zeros((384, 384), jnp.float32).at[:372, :].set(w_ih.T)
    upd = pl.pallas_call(
        _gru_body,
        grid=(N // BR,),
        in_specs=[
            pl.BlockSpec((BR, XW), lambda i: (i, 0)),
            pl.BlockSpec((384, 384), lambda i: (0, 0)),
            pl.BlockSpec((128, 384), lambda i: (0, 0)),
            pl.BlockSpec((1, 384), lambda i: (0, 0)),
            pl.BlockSpec((1, 384), lambda i: (0, 0)),
        ],
        out_specs=pl.BlockSpec((BR, 128), lambda i: (i, 0)),
        out_shape=jax.ShapeDtypeStruct((N, 128), jnp.float32),
    )(table, wih_pad, w_hh.T, b_ih.reshape(1, 384), b_hh.reshape(1, 384))

    restored = _gather_kernel(upd, ids_g)

    h_out = pl.pallas_call(
        _final_body,
        grid=(N // BR,),
        in_specs=[
            pl.BlockSpec((BR, 128), lambda i: (i, 0)),
            pl.BlockSpec((BR, 256), lambda i: (i, 0)),
            pl.BlockSpec((256, 128), lambda i: (0, 0)),
            pl.BlockSpec((1, 128), lambda i: (0, 0)),
        ],
        out_specs=pl.BlockSpec((BR, 128), lambda i: (i, 0)),
        out_shape=jax.ShapeDtypeStruct((N, 128), jnp.float32),
    )(restored, h, proj_w.T, proj_b.reshape(1, 128))

    nd = 50000
    last_updated_nid = all_ids[:nd] + (num_dst_nodes - nd)
    return last_updated_nid, restored[:nd], ts[:nd], h_out


# speedup vs baseline: 1.3474x; 1.2632x over previous
import functools

import jax
import jax.numpy as jnp
from jax import lax
from jax.experimental import pallas as pl
from jax.experimental.pallas import tpu as pltpu
from jax.experimental.pallas import tpu_sc as plsc

N = 100000
XW = 512
CNT_COL = 383
NPASS = 16
CH = 125
NSUB = 16
NCORE = 2
ROWS_PER_TILE = N // NSUB
NCHUNK_S = ROWS_PER_TILE // CH
ROWS_PER_W = N // (NSUB * NCORE)
NCHUNK_G = ROWS_PER_W // CH
ZROWS = 625
NBUF = 5
NGROUP = NCHUNK_S // NBUF
BR = 1000


def _build_body(mi_ref, ts_ref, mts_ref, tw_ref, tb_ref, mem_ref, x_ref):
    dt = ts_ref[...] - mts_ref[...]
    tf = jnp.cos(dt * tw_ref[...] + tb_ref[...])
    br = mi_ref.shape[0]
    x_ref[...] = jnp.concatenate([
        mi_ref[...], tf,
        jnp.zeros((br, 11), jnp.float32),
        jnp.ones((br, 1), jnp.float32),
        mem_ref[...],
    ], axis=1)


def _gru_body(t_ref, wih_ref, whh_ref, bih_ref, bhh_ref, upd_ref):
    tb = t_ref[...]
    xa = tb[:, :384]
    ma = tb[:, 384:]
    cnt = tb[:, CNT_COL:CNT_COL + 1]
    hprev = ma / jnp.maximum(cnt, 1.0)
    gi = jnp.dot(xa, wih_ref[...], preferred_element_type=jnp.float32) + bih_ref[...]
    gh = jnp.dot(hprev, whh_ref[...], preferred_element_type=jnp.float32) + bhh_ref[...]
    r = jax.nn.sigmoid(gi[:, :128] + gh[:, :128])
    z = jax.nn.sigmoid(gi[:, 128:256] + gh[:, 128:256])
    n = jnp.tanh(gi[:, 256:] + r * gh[:, 256:])
    upd_ref[...] = (1.0 - z) * n + z * hprev


def _final_body(r_ref, h_ref, pw_ref, pb_ref, o_ref):
    o_ref[...] = (r_ref[...]
                  + jnp.dot(h_ref[...], pw_ref[...], preferred_element_type=jnp.float32)
                  + pb_ref[...])


_sc_mesh = plsc.VectorSubcoreMesh(core_axis_name="c", subcore_axis_name="s")
_sc_params = pltpu.CompilerParams(use_tc_tiling_on_sc=False)


@functools.partial(
    pl.kernel,
    out_type=jax.ShapeDtypeStruct((N, XW), jnp.float32),
    mesh=_sc_mesh,
    compiler_params=_sc_params,
    scratch_types=[
        pltpu.VMEM_SHARED((N, 16), jnp.float32),
        pltpu.VMEM((NCHUNK_S, CH), jnp.int32),
        pltpu.VMEM((NBUF, CH, 16), jnp.float32),
        pltpu.VMEM((ZROWS, 16), jnp.float32),
        pltpu.SemaphoreType.DMA((NBUF,)),
        pltpu.SemaphoreType.DMA((NBUF,)),
    ],
)
def _scatter_kernel(x_hbm, ids_hbm, table_hbm, acc, ids_v, buf, zbuf, gsem, ssem):
    c = lax.axis_index("c")
    s = lax.axis_index("s")
    row0 = s * ROWS_PER_TILE
    pltpu.sync_copy(ids_hbm.at[s], ids_v)

    def zrow(i, carry):
        zbuf[i, :] = jnp.zeros((16,), jnp.float32)
        return carry
    lax.fori_loop(0, ZROWS, zrow, 0)

    def xsrc(j, col0):
        return x_hbm.at[pl.ds(row0 + j * CH, CH), pl.ds(col0, 16)]

    for p in range(NPASS):
        col0 = (p * NCORE + c) * 16

        def zcopy(i, carry):
            pltpu.sync_copy(zbuf, acc.at[pl.ds(row0 + i * ZROWS, ZROWS), :])
            return carry
        lax.fori_loop(0, ROWS_PER_TILE // ZROWS, zcopy, 0)
        plsc.subcore_barrier()

        for b in range(NBUF):
            pltpu.async_copy(xsrc(b, col0), buf.at[b], gsem.at[b])

        def group(g, carry):
            for b in range(NBUF):
                j = g * NBUF + b
                pltpu.make_async_copy(xsrc(j, col0), buf.at[b], gsem.at[b]).wait()
                pltpu.async_copy(buf.at[b], acc.at[ids_v.at[j]], ssem.at[b],
                                 add=True)
            for b in range(NBUF):
                j = g * NBUF + b
                pltpu.make_async_copy(buf.at[b], acc.at[ids_v.at[j]],
                                      ssem.at[b]).wait()
                jn = j + NBUF

                @pl.when(jn < NCHUNK_S)
                def _():
                    pltpu.async_copy(xsrc(jn, col0), buf.at[b], gsem.at[b])
            return carry
        lax.fori_loop(0, NGROUP, group, 0)
        plsc.subcore_barrier()

        pltpu.sync_copy(
            acc.at[pl.ds(row0, ROWS_PER_TILE), :],
            table_hbm.at[pl.ds(row0, ROWS_PER_TILE), pl.ds(col0, 16)])


@functools.partial(
    pl.kernel,
    out_type=jax.ShapeDtypeStruct((N, 128), jnp.float32),
    mesh=_sc_mesh,
    compiler_params=_sc_params,
    scratch_types=[
        pltpu.VMEM((NCHUNK_G, CH), jnp.int32),
        pltpu.VMEM((CH, 128), jnp.float32),
        pltpu.SemaphoreType.DMA,
    ],
)
def _gather_kernel(upd_hbm, ids_hbm, out_hbm, ids_v, rows_v, sem):
    c = lax.axis_index("c")
    s = lax.axis_index("s")
    w = s * NCORE + c
    chunk0 = w * NCHUNK_G
    pltpu.sync_copy(ids_hbm.at[w], ids_v)

    def chunk(j, carry):
        pltpu.async_copy(upd_hbm.at[ids_v.at[j]], rows_v, sem).wait()
        pltpu.sync_copy(rows_v, out_hbm.at[pl.ds((chunk0 + j) * CH, CH), :])
        return carry
    lax.fori_loop(0, NCHUNK_G, chunk, 0)


def kernel(all_ids, mem_input, ts, mem_ts, mem, h, num_dst_nodes,
           time_w, time_b, w_ih, w_hh, b_ih, b_hh, proj_w, proj_b):
    ids_i32 = all_ids.astype(jnp.int32)
    ids_s = ids_i32.reshape(NSUB, NCHUNK_S, CH)
    ids_g = ids_i32.reshape(NSUB * NCORE, NCHUNK_G, CH)

    x = pl.pallas_call(
        _build_body,
        grid=(N // BR,),
        in_specs=[
            pl.BlockSpec((BR, 272), lambda i: (i, 0)),
            pl.BlockSpec((BR, 1), lambda i: (i, 0)),
            pl.BlockSpec((BR, 1), lambda i: (i, 0)),
            pl.BlockSpec((1, 100), lambda i: (0, 0)),
            pl.BlockSpec((1, 100), lambda i: (0, 0)),
            pl.BlockSpec((BR, 128), lambda i: (i, 0)),
        ],
        out_specs=pl.BlockSpec((BR, XW), lambda i: (i, 0)),
        out_shape=jax.ShapeDtypeStruct((N, XW), jnp.float32),
    )(mem_input, ts.reshape(N, 1), mem_ts.reshape(N, 1),
      time_w.reshape(1, 100), time_b.reshape(1, 100), mem)

    table = _scatter_kernel(x, ids_s)

    wih_pad = jnp.zeros((384, 384), jnp.float32).at[:372, :].set(w_ih.T)
    upd = pl.pallas_call(
        _gru_body,
        grid=(N // BR,),
        in_specs=[
            pl.BlockSpec((BR, XW), lambda i: (i, 0)),
            pl.BlockSpec((384, 384), lambda i: (0, 0)),
            pl.BlockSpec((128, 384), lambda i: (0, 0)),
            pl.BlockSpec((1, 384), lambda i: (0, 0)),
            pl.BlockSpec((1, 384), lambda i: (0, 0)),
        ],
        out_specs=pl.BlockSpec((BR, 128), lambda i: (i, 0)),
        out_shape=jax.ShapeDtypeStruct((N, 128), jnp.float32),
    )(table, wih_pad, w_hh.T, b_ih.reshape(1, 384), b_hh.reshape(1, 384))

    restored = _gather_kernel(upd, ids_g)

    h_out = pl.pallas_call(
        _final_body,
        grid=(N // BR,),
        in_specs=[
            pl.BlockSpec((BR, 128), lambda i: (i, 0)),
            pl.BlockSpec((BR, 256), lambda i: (i, 0)),
            pl.BlockSpec((256, 128), lambda i: (0, 0)),
            pl.BlockSpec((1, 128), lambda i: (0, 0)),
        ],
        out_specs=pl.BlockSpec((BR, 128), lambda i: (i, 0)),
        out_shape=jax.ShapeDtypeStruct((N, 128), jnp.float32),
    )(restored, h, proj_w.T, proj_b.reshape(1, 128))

    nd = 50000
    last_updated_nid = all_ids[:nd] + (num_dst_nodes - nd)
    return last_updated_nid, restored[:nd], ts[:nd], h_out
